# baseline (speedup 1.0000x reference)
import functools

import jax
import jax.numpy as jnp
from jax import lax
from jax.experimental import pallas as pl
from jax.experimental.pallas import tpu as pltpu
from jax.experimental.pallas import tpu_sc as plsc

N_ENT = 1000000
N_REL = 1000
EMB = 64
BATCH = 16384

NC = 2
NS = 16
L = 16
NW = NC * NS
BPW = BATCH // NW
CHUNK = 128
NCHUNK = BPW // CHUNK


def _body(s_hbm, p_hbm, o_hbm, ent_hbm, rel_hbm, out_hbm,
          si, pi, oi, se, pe, oe, outv, sem):
    wid = lax.axis_index("s") * NC + lax.axis_index("c")
    base = wid * BPW

    idx_copies = []
    for j in range(NCHUNK):
        off = base + j * CHUNK
        idx_copies.append(pltpu.async_copy(s_hbm.at[pl.ds(off, CHUNK)], si.at[j], sem))
        idx_copies.append(pltpu.async_copy(p_hbm.at[pl.ds(off, CHUNK)], pi.at[j], sem))
        idx_copies.append(pltpu.async_copy(o_hbm.at[pl.ds(off, CHUNK)], oi.at[j], sem))
    for c in idx_copies:
        c.wait()

    gathers = []
    for j in range(NCHUNK):
        r = pl.ds(j * CHUNK, CHUNK)
        gathers.append(pltpu.async_copy(ent_hbm.at[si.at[j]], se.at[r, :], sem))
        gathers.append(pltpu.async_copy(rel_hbm.at[pi.at[j]], pe.at[r, :], sem))
        gathers.append(pltpu.async_copy(ent_hbm.at[oi.at[j]], oe.at[r, :], sem))
    for c in gathers:
        c.wait()

    lane = lax.iota(jnp.int32, L)

    def group(g, _):
        rows = g * L + lane
        acc = jnp.zeros((L,), jnp.float32)
        for j in range(EMB):
            cols = jnp.full((L,), j, jnp.int32)
            a = plsc.load_gather(se, [rows, cols])
            b = plsc.load_gather(pe, [rows, cols])
            c = plsc.load_gather(oe, [rows, cols])
            acc = acc + a * b * c
        plsc.store_scatter(outv, [rows], acc)
        return _

    lax.fori_loop(0, BPW // L, group, None)

    pltpu.sync_copy(outv, out_hbm.at[pl.ds(base, BPW)])


@jax.jit
def _distmult(s, p, o, entities, relations):
    mesh = plsc.VectorSubcoreMesh(core_axis_name="c", subcore_axis_name="s")
    kern = functools.partial(
        pl.kernel,
        mesh=mesh,
        compiler_params=pltpu.CompilerParams(
            needs_layout_passes=False, use_tc_tiling_on_sc=False),
        out_type=jax.ShapeDtypeStruct((BATCH,), jnp.float32),
        scratch_types=[
            pltpu.VMEM((NCHUNK, CHUNK), jnp.int32),
            pltpu.VMEM((NCHUNK, CHUNK), jnp.int32),
            pltpu.VMEM((NCHUNK, CHUNK), jnp.int32),
            pltpu.VMEM((BPW, EMB), jnp.float32),
            pltpu.VMEM((BPW, EMB), jnp.float32),
            pltpu.VMEM((BPW, EMB), jnp.float32),
            pltpu.VMEM((BPW,), jnp.float32),
            pltpu.SemaphoreType.DMA,
        ],
    )(_body)
    return kern(s, p, o, entities, relations)


def kernel(s, p, o, entities, relations):
    return _distmult(s, p, o, entities, relations)


# speedup vs baseline: 1.0066x; 1.0066x over previous
import functools

import jax
import jax.numpy as jnp
from jax import lax
from jax.experimental import pallas as pl
from jax.experimental.pallas import tpu as pltpu
from jax.experimental.pallas import tpu_sc as plsc

N_ENT = 1000000
N_REL = 1000
EMB = 64
BATCH = 16384

NC = 2
NS = 16
L = 16
NW = NC * NS
BPW = BATCH // NW
CHUNK = 128
NCHUNK = BPW // CHUNK
HALF = BPW // 2


def _body(s_hbm, p_hbm, o_hbm, ent_hbm, rel_hbm, out_hbm,
          si, pi, oi, ssh, psh, osh, sb, pb, ob, se, pe, oe, outv, sem):
    wid = lax.axis_index("s") * NC + lax.axis_index("c")
    base = wid * BPW

    idx_copies = []
    for j in range(NCHUNK):
        off = base + j * CHUNK
        idx_copies.append(pltpu.async_copy(s_hbm.at[pl.ds(off, CHUNK)], si.at[j], sem))
        idx_copies.append(pltpu.async_copy(p_hbm.at[pl.ds(off, CHUNK)], pi.at[j], sem))
        idx_copies.append(pltpu.async_copy(o_hbm.at[pl.ds(off, CHUNK)], oi.at[j], sem))
    for c in idx_copies:
        c.wait()

    for c in range(NCHUNK):
        for k in range(CHUNK // L):
            sl = pl.ds(k * L, L)
            sv = si[c, sl]
            pv = pi[c, sl]
            ov = oi[c, sl]
            ssh[c, sl] = lax.shift_right_logical(sv, 1)
            psh[c, sl] = lax.shift_right_logical(pv, 1)
            osh[c, sl] = lax.shift_right_logical(ov, 1)
            fl = pl.ds(c * CHUNK + k * L, L)
            sb[fl] = lax.shift_left(jnp.bitwise_and(sv, 1), 6)
            pb[fl] = lax.shift_left(jnp.bitwise_and(pv, 1), 6)
            ob[fl] = lax.shift_left(jnp.bitwise_and(ov, 1), 6)

    lane = lax.iota(jnp.int32, L)

    for half in range(2):
        gathers = []
        for c in range(2):
            ch = half * 2 + c
            r = pl.ds(c * CHUNK, CHUNK)
            gathers.append(pltpu.async_copy(ent_hbm.at[ssh.at[ch]], se.at[r, :], sem))
            gathers.append(pltpu.async_copy(rel_hbm.at[psh.at[ch]], pe.at[r, :], sem))
            gathers.append(pltpu.async_copy(ent_hbm.at[osh.at[ch]], oe.at[r, :], sem))
        for c in gathers:
            c.wait()

        def group(g, _, half=half):
            rows = g * L + lane
            fl = pl.ds(half * HALF + g * L, L)
            cs0 = sb[fl]
            cp0 = pb[fl]
            co0 = ob[fl]
            acc = jnp.zeros((L,), jnp.float32)
            for j in range(EMB):
                a = plsc.load_gather(se, [rows, cs0 + j])
                b = plsc.load_gather(pe, [rows, cp0 + j])
                c = plsc.load_gather(oe, [rows, co0 + j])
                acc = acc + a * b * c
            plsc.store_scatter(outv, [half * HALF + rows], acc)
            return _

        lax.fori_loop(0, HALF // L, group, None)

    pltpu.sync_copy(outv, out_hbm.at[pl.ds(base, BPW)])


@jax.jit
def _distmult(s, p, o, entities, relations):
    mesh = plsc.VectorSubcoreMesh(core_axis_name="c", subcore_axis_name="s")
    kern = functools.partial(
        pl.kernel,
        mesh=mesh,
        compiler_params=pltpu.CompilerParams(
            needs_layout_passes=False, use_tc_tiling_on_sc=True),
        out_type=jax.ShapeDtypeStruct((BATCH,), jnp.float32),
        scratch_types=[
            pltpu.VMEM((NCHUNK, CHUNK), jnp.int32),
            pltpu.VMEM((NCHUNK, CHUNK), jnp.int32),
            pltpu.VMEM((NCHUNK, CHUNK), jnp.int32),
            pltpu.VMEM((NCHUNK, CHUNK), jnp.int32),
            pltpu.VMEM((NCHUNK, CHUNK), jnp.int32),
            pltpu.VMEM((NCHUNK, CHUNK), jnp.int32),
            pltpu.VMEM((BPW,), jnp.int32),
            pltpu.VMEM((BPW,), jnp.int32),
            pltpu.VMEM((BPW,), jnp.int32),
            pltpu.VMEM((HALF, 2 * EMB), jnp.float32),
            pltpu.VMEM((HALF, 2 * EMB), jnp.float32),
            pltpu.VMEM((HALF, 2 * EMB), jnp.float32),
            pltpu.VMEM((BPW,), jnp.float32),
            pltpu.SemaphoreType.DMA,
        ],
    )(_body)
    ent2 = entities.reshape(N_ENT // 2, 2 * EMB)
    rel2 = relations.reshape(N_REL // 2, 2 * EMB)
    return kern(s, p, o, ent2, rel2)


def kernel(s, p, o, entities, relations):
    return _distmult(s, p, o, entities, relations)
